# Initial kernel scaffold; baseline (speedup 1.0000x reference)
import jax
import jax.numpy as jnp
from jax.experimental import pallas as pl


def kernel(edge_attr, edge_attr2, edge_index, edge_index2, num_nodes):
    raise NotImplementedError("write your pallas kernel here")


# speedup vs baseline: 5.2893x; 5.2893x over previous
import functools

import jax
import jax.numpy as jnp
from jax import lax
from jax.experimental import pallas as pl
from jax.experimental.pallas import tpu as pltpu
from jax.experimental.pallas import tpu_sc as plsc

NUM_NODES = 100000
D = 16
NC = 2
NS = 16
NW = NC * NS
IDX_W = 128
CHUNK_ROWS = 8
CHUNK_E = CHUNK_ROWS * IDX_W
ZROWS = 6256
ACC_ROWS = NS * ZROWS
RD_ROWS = ZROWS


N_OUT_ROWS = NUM_NODES * D // 128
N_ACC_OUT_ROWS = ACC_ROWS * D // 128


def _add_body(p_ref, o_ref):
    o_ref[...] = p_ref[0, :N_OUT_ROWS] + p_ref[1, :N_OUT_ROWS]


@functools.lru_cache(maxsize=None)
def _sc_scatter(E, cpw):
    @functools.partial(
        pl.kernel,
        out_type=jax.ShapeDtypeStruct((NC, ACC_ROWS, D), jnp.float32),
        mesh=plsc.VectorSubcoreMesh(core_axis_name="c", subcore_axis_name="s"),
        compiler_params=pltpu.CompilerParams(use_tc_tiling_on_sc=False),
        scratch_types=[
            pltpu.VMEM_SHARED((ACC_ROWS, D), jnp.float32),
            pltpu.VMEM((CHUNK_ROWS, IDX_W), jnp.int32),
            pltpu.VMEM((CHUNK_E, D), jnp.float32),
        ],
    )
    def k(idx1_hbm, attr1_hbm, idx2_hbm, attr2_hbm, zeros_hbm, out_hbm,
          acc, idx_v, rows_v):
        cid = lax.axis_index("c")
        sid = lax.axis_index("s")
        wid = sid * NC + cid

        pltpu.sync_copy(zeros_hbm, acc.at[pl.ds(sid * ZROWS, ZROWS)])
        plsc.subcore_barrier()

        def run(idx_hbm, attr_hbm):
            def body(c, carry):
                row0 = (wid * cpw + c) * CHUNK_ROWS
                eb = jnp.minimum(row0 * IDX_W, E - CHUNK_E)
                pltpu.sync_copy(idx_hbm.at[pl.ds(row0, CHUNK_ROWS)], idx_v)
                pltpu.sync_copy(attr_hbm.at[pl.ds(eb, CHUNK_E)], rows_v)
                for j in range(CHUNK_ROWS):
                    pltpu.sync_copy(rows_v.at[pl.ds(j * IDX_W, IDX_W)],
                                    acc.at[idx_v.at[j]], add=True)
                return carry
            lax.fori_loop(0, cpw, body, 0)

        run(idx1_hbm, attr1_hbm)
        run(idx2_hbm, attr2_hbm)
        plsc.subcore_barrier()

        pltpu.sync_copy(acc.at[pl.ds(sid * RD_ROWS, RD_ROWS)],
                        out_hbm.at[cid, pl.ds(sid * RD_ROWS, RD_ROWS)])

    return k


@functools.lru_cache(maxsize=None)
def _tc_add():
    return pl.pallas_call(
        _add_body,
        out_shape=jax.ShapeDtypeStruct((N_OUT_ROWS, 128), jnp.float32),
    )


def kernel(edge_attr, edge_attr2, edge_index, edge_index2, num_nodes):
    E = edge_attr.shape[0]
    rows = -(-E // IDX_W)
    cpw = -(-rows // (CHUNK_ROWS * NW))
    rows_pad = cpw * CHUNK_ROWS * NW
    e_pad = rows_pad * IDX_W

    def pad_idx(ei):
        idx = ei[0].astype(jnp.int32)
        pad = jnp.full((e_pad - E,), NUM_NODES, jnp.int32)
        return jnp.concatenate([idx, pad]).reshape(rows_pad, IDX_W)

    idx1 = pad_idx(edge_index)
    idx2 = pad_idx(edge_index2)
    zeros = jnp.zeros((ZROWS, D), jnp.float32)

    partial = _sc_scatter(E, cpw)(idx1, edge_attr, idx2, edge_attr2, zeros)
    out = _tc_add()(partial.reshape(2, N_ACC_OUT_ROWS, 128))
    return out.reshape(NUM_NODES, D)


# speedup vs baseline: 11.9291x; 2.2553x over previous
import functools

import jax
import jax.numpy as jnp
from jax import lax
from jax.experimental import pallas as pl
from jax.experimental.pallas import tpu as pltpu
from jax.experimental.pallas import tpu_sc as plsc

NUM_NODES = 100000
D = 16
NC = 2
NS = 16
L = 16
KB = 25
K = KB * 128


def _add_body(p_ref, o_ref):
    o_ref[...] = p_ref[0] + p_ref[1]


@functools.lru_cache(maxsize=None)
def _sc_scatter(E):
    nblk = E // 128
    blk_sc = nblk // NC
    iters = blk_sc // KB
    steps = iters // 2
    e_sc = E // NC
    groups = K // L

    @functools.partial(
        pl.kernel,
        out_type=jax.ShapeDtypeStruct((NC, NS, NUM_NODES), jnp.float32),
        mesh=plsc.VectorSubcoreMesh(core_axis_name="c", subcore_axis_name="s"),
        compiler_params=pltpu.CompilerParams(
            use_tc_tiling_on_sc=False, needs_layout_passes=False),
        scratch_types=[
            pltpu.VMEM((NUM_NODES,), jnp.float32),
            pltpu.VMEM((2, K), jnp.int32),
            pltpu.VMEM((2, KB, 128), jnp.float32),
            pltpu.SemaphoreType.DMA((2,)),
        ],
    )
    def k(idx1_hbm, attr1_hbm, idx2_hbm, attr2_hbm, out_hbm,
          acc, idx_v, vals_v, sems):
        cid = lax.axis_index("c")
        sid = lax.axis_index("s")
        fi = sid // 8
        f8 = sid % 8

        zeros16 = jnp.zeros((L,), jnp.float32)

        def zbody(t, carry):
            acc[pl.ds(t * L, L)] = zeros16
            return carry
        lax.fori_loop(0, NUM_NODES // L, zbody, 0)

        def issue(idx_hbm, attr_hbm, it, b):
            pltpu.async_copy(
                idx_hbm.at[pl.ds(cid * e_sc + it * K, K)],
                idx_v.at[b], sems.at[b])
            pltpu.async_copy(
                attr_hbm.at[fi, pl.ds(cid * blk_sc + it * KB, KB), f8],
                vals_v.at[b], sems.at[b])

        def drain(idx_hbm, attr_hbm, b):
            pltpu.make_async_copy(
                idx_hbm.at[pl.ds(0, K)], idx_v.at[b], sems.at[b]).wait()
            pltpu.make_async_copy(
                attr_hbm.at[0, pl.ds(0, KB), 0], vals_v.at[b],
                sems.at[b]).wait()

        def run(idx_hbm, attr_hbm):
            issue(idx_hbm, attr_hbm, 0, 0)
            issue(idx_hbm, attr_hbm, 1, 1)

            def body(step, carry):
                for b in (0, 1):
                    drain(idx_hbm, attr_hbm, b)
                    for g in range(groups):
                        gi = idx_v[b, pl.ds(g * L, L)]
                        gv = vals_v[b, g // 8, pl.ds((g % 8) * L, L)]
                        plsc.addupdate_scatter(acc, [gi], gv)
                    @pl.when(step < steps - 1)
                    def _():
                        issue(idx_hbm, attr_hbm, step * 2 + 2 + b, b)
                return carry
            lax.fori_loop(0, steps, body, 0)

        run(idx1_hbm, attr1_hbm)
        run(idx2_hbm, attr2_hbm)

        pltpu.sync_copy(acc, out_hbm.at[cid, sid])

    return k


@functools.lru_cache(maxsize=None)
def _tc_add():
    blk = 6400
    return pl.pallas_call(
        _add_body,
        grid=(pl.cdiv(NUM_NODES, blk),),
        in_specs=[pl.BlockSpec((2, D, blk), lambda i: (0, 0, i))],
        out_specs=pl.BlockSpec((D, blk), lambda i: (0, i)),
        out_shape=jax.ShapeDtypeStruct((D, NUM_NODES), jnp.float32),
    )


def kernel(edge_attr, edge_attr2, edge_index, edge_index2, num_nodes):
    E = edge_attr.shape[0]
    nblk = E // 128

    def as_storage_view(attr):
        return attr.T.reshape(2, 8, nblk, 128).transpose(0, 2, 1, 3)

    idx1 = edge_index[0].astype(jnp.int32)
    idx2 = edge_index2[0].astype(jnp.int32)
    attr1 = as_storage_view(edge_attr)
    attr2 = as_storage_view(edge_attr2)

    partial = _sc_scatter(E)(idx1, attr1, idx2, attr2)
    out_t = _tc_add()(partial)
    return out_t.T


# speedup vs baseline: 13.9437x; 1.1689x over previous
import functools

import jax
import jax.numpy as jnp
from jax import lax
from jax.experimental import pallas as pl
from jax.experimental.pallas import tpu as pltpu
from jax.experimental.pallas import tpu_sc as plsc

NUM_NODES = 100000
D = 16
NC = 2
NS = 16
L = 16
KB = 50
K = KB * 128


def _add_body(p_ref, o_ref):
    o_ref[...] = p_ref[0] + p_ref[1]


@functools.lru_cache(maxsize=None)
def _sc_scatter(E):
    nblk = E // 128
    blk_sc = nblk // NC
    iters = blk_sc // KB
    steps = iters // 2
    e_sc = E // NC
    groups = K // L

    @functools.partial(
        pl.kernel,
        out_type=jax.ShapeDtypeStruct((NC, NS, NUM_NODES), jnp.float32),
        mesh=plsc.VectorSubcoreMesh(core_axis_name="c", subcore_axis_name="s"),
        compiler_params=pltpu.CompilerParams(
            use_tc_tiling_on_sc=False, needs_layout_passes=False),
        scratch_types=[
            pltpu.VMEM((NUM_NODES,), jnp.float32),
            pltpu.VMEM((2, K), jnp.int32),
            pltpu.VMEM((2, KB, 128), jnp.float32),
            pltpu.SemaphoreType.DMA((2,)),
        ],
    )
    def k(idx1_hbm, attr1_hbm, idx2_hbm, attr2_hbm, out_hbm,
          acc, idx_v, vals_v, sems):
        cid = lax.axis_index("c")
        sid = lax.axis_index("s")
        fi = sid // 8
        f8 = sid % 8

        zeros16 = jnp.zeros((L,), jnp.float32)

        def zbody(t, carry):
            acc[pl.ds(t * L, L)] = zeros16
            return carry
        lax.fori_loop(0, NUM_NODES // L, zbody, 0)

        def issue(idx_hbm, attr_hbm, it, b):
            pltpu.async_copy(
                idx_hbm.at[pl.ds(cid * e_sc + it * K, K)],
                idx_v.at[b], sems.at[b])
            pltpu.async_copy(
                attr_hbm.at[fi, pl.ds(cid * blk_sc + it * KB, KB), f8],
                vals_v.at[b], sems.at[b])

        def drain(idx_hbm, attr_hbm, b):
            pltpu.make_async_copy(
                idx_hbm.at[pl.ds(0, K)], idx_v.at[b], sems.at[b]).wait()
            pltpu.make_async_copy(
                attr_hbm.at[0, pl.ds(0, KB), 0], vals_v.at[b],
                sems.at[b]).wait()

        def run(idx_hbm, attr_hbm):
            issue(idx_hbm, attr_hbm, 0, 0)
            issue(idx_hbm, attr_hbm, 1, 1)

            def body(step, carry):
                for b in (0, 1):
                    drain(idx_hbm, attr_hbm, b)
                    gi = idx_v[b, pl.ds(0, L)]
                    gv = vals_v[b, 0, pl.ds(0, L)]
                    for g in range(1, groups + 1):
                        if g < groups:
                            ni = idx_v[b, pl.ds(g * L, L)]
                            nv = vals_v[b, g // 8, pl.ds((g % 8) * L, L)]
                        plsc.addupdate_scatter(acc, [gi], gv)
                        if g < groups:
                            gi, gv = ni, nv
                    @pl.when(step < steps - 1)
                    def _():
                        issue(idx_hbm, attr_hbm, step * 2 + 2 + b, b)
                return carry
            lax.fori_loop(0, steps, body, 0)

        run(idx1_hbm, attr1_hbm)
        run(idx2_hbm, attr2_hbm)

        pltpu.sync_copy(acc, out_hbm.at[cid, sid])

    return k


@functools.lru_cache(maxsize=None)
def _tc_add():
    blk = 6400
    return pl.pallas_call(
        _add_body,
        grid=(pl.cdiv(NUM_NODES, blk),),
        in_specs=[pl.BlockSpec((2, D, blk), lambda i: (0, 0, i))],
        out_specs=pl.BlockSpec((D, blk), lambda i: (0, i)),
        out_shape=jax.ShapeDtypeStruct((D, NUM_NODES), jnp.float32),
    )


def kernel(edge_attr, edge_attr2, edge_index, edge_index2, num_nodes):
    E = edge_attr.shape[0]
    nblk = E // 128

    def as_storage_view(attr):
        return attr.T.reshape(2, 8, nblk, 128).transpose(0, 2, 1, 3)

    idx1 = edge_index[0].astype(jnp.int32)
    idx2 = edge_index2[0].astype(jnp.int32)
    attr1 = as_storage_view(edge_attr)
    attr2 = as_storage_view(edge_attr2)

    partial = _sc_scatter(E)(idx1, attr1, idx2, attr2)
    out_t = _tc_add()(partial)
    return out_t.T
